# baseline (speedup 1.0000x reference)
import functools

import jax
import jax.numpy as jnp
from jax import lax
from jax.experimental import pallas as pl
from jax.experimental.pallas import tpu as pltpu
from jax.experimental.pallas import tpu_sc as plsc

B = 16
TOTAL = 16384
D = 256
SEG = TOTAL // B
NC = 2
NS = 16
NW = NC * NS
ROWS_W = TOTAL // NW
CH = 128
NCH = ROWS_W // CH
LANES = 16
G = D // LANES

_sc_mesh = plsc.VectorSubcoreMesh(core_axis_name="c", subcore_axis_name="s")


@functools.partial(
    pl.kernel,
    out_type=(
        jax.ShapeDtypeStruct((TOTAL, 4 * D), jnp.float32),
        jax.ShapeDtypeStruct((NW, 3 * D), jnp.float32),
    ),
    mesh=_sc_mesh,
    scratch_types=[
        pltpu.VMEM((CH, D), jnp.float32),
        pltpu.VMEM((3 * D,), jnp.float32),
    ],
)
def _sc_stats(x_hbm, buf_hbm, part_hbm, xv, pv):
    c = lax.axis_index("c")
    sub = lax.axis_index("s")
    w = c * NS + sub
    row0 = w * ROWS_W

    zero = jnp.zeros((LANES,), jnp.float32)
    pinf = jnp.full((LANES,), jnp.inf, jnp.float32)
    ninf = jnp.full((LANES,), -jnp.inf, jnp.float32)
    carry = (
        tuple(zero for _ in range(G)),
        tuple(pinf for _ in range(G)),
        tuple(ninf for _ in range(G)),
    )

    def row_body(r, acc):
        sums, mns, mxs = acc
        new_s, new_n, new_x = [], [], []
        for g in range(G):
            v = xv[r, pl.ds(g * LANES, LANES)]
            new_s.append(sums[g] + v)
            new_n.append(jnp.minimum(mns[g], v))
            new_x.append(jnp.maximum(mxs[g], v))
        return (tuple(new_s), tuple(new_n), tuple(new_x))

    for k in range(NCH):
        r0 = row0 + k * CH
        pltpu.sync_copy(x_hbm.at[pl.ds(r0, CH)], xv)
        pltpu.sync_copy(xv, buf_hbm.at[pl.ds(r0, CH), pl.ds(3 * D, D)])
        carry = lax.fori_loop(0, CH, row_body, carry)

    sums, mns, mxs = carry
    for g in range(G):
        pv[pl.ds(g * LANES, LANES)] = sums[g]
        pv[pl.ds(D + g * LANES, LANES)] = mns[g]
        pv[pl.ds(2 * D + g * LANES, LANES)] = mxs[g]
    pltpu.sync_copy(pv, part_hbm.at[w])


def _asm_kernel(inv_ref, part_ref, buf_ref, out_ref):
    del buf_ref
    i = pl.program_id(0)
    j = pl.program_id(1)
    p = part_ref[0]
    a = p[0:1]
    b = p[1:2]
    mean = (a + b) * inv_ref[i]
    mn = jnp.minimum(a, b)
    mx = jnp.maximum(a, b)
    sel = jnp.where(j == 0, mean, jnp.where(j == 1, mn, mx))
    out_ref[...] = jnp.broadcast_to(sel, (SEG, D))


def kernel(x_data, row_splits):
    counts = (row_splits[1:] - row_splits[:-1]).astype(jnp.float32)
    inv_counts = 1.0 / counts
    buf, part = _sc_stats(x_data)
    part3 = part.reshape(B, 2, 3 * D)
    return pl.pallas_call(
        _asm_kernel,
        grid_spec=pltpu.PrefetchScalarGridSpec(
            num_scalar_prefetch=1,
            grid=(B, 3),
            in_specs=[
                pl.BlockSpec((1, 2, D), lambda i, j, *_: (i, 0, j)),
                pl.BlockSpec(memory_space=pltpu.MemorySpace.HBM),
            ],
            out_specs=pl.BlockSpec((SEG, D), lambda i, j, *_: (i, j)),
        ),
        out_shape=jax.ShapeDtypeStruct((TOTAL, 4 * D), jnp.float32),
        input_output_aliases={2: 0},
    )(inv_counts, part3, buf)


# speedup vs baseline: 1.1919x; 1.1919x over previous
import functools

import jax
import jax.numpy as jnp
from jax import lax
from jax.experimental import pallas as pl
from jax.experimental.pallas import tpu as pltpu
from jax.experimental.pallas import tpu_sc as plsc

B = 16
TOTAL = 16384
D = 256
SEG = TOTAL // B
NC = 2
NS = 16
NW = NC * NS
ROWS_W = TOTAL // NW
CH = 128
NCH = ROWS_W // CH
LANES = 16
G = D // LANES

_sc_mesh = plsc.VectorSubcoreMesh(core_axis_name="c", subcore_axis_name="s")


@functools.partial(
    pl.kernel,
    out_type=jax.ShapeDtypeStruct((NW, 3 * D), jnp.float32),
    mesh=_sc_mesh,
    scratch_types=[
        pltpu.VMEM((CH, D), jnp.float32),
        pltpu.VMEM((CH, D), jnp.float32),
        pltpu.VMEM((3 * D,), jnp.float32),
        pltpu.SemaphoreType.DMA,
        pltpu.SemaphoreType.DMA,
    ],
)
def _sc_stats(x_hbm, part_hbm, xv0, xv1, pv, sem0, sem1):
    c = lax.axis_index("c")
    sub = lax.axis_index("s")
    w = c * NS + sub
    row0 = w * ROWS_W

    bufs = (xv0, xv1)
    sems = (sem0, sem1)

    zero = jnp.zeros((LANES,), jnp.float32)
    pinf = jnp.full((LANES,), jnp.inf, jnp.float32)
    ninf = jnp.full((LANES,), -jnp.inf, jnp.float32)
    carry = (
        tuple(zero for _ in range(G)),
        tuple(pinf for _ in range(G)),
        tuple(ninf for _ in range(G)),
    )

    handles = [None, None]
    handles[0] = pltpu.async_copy(x_hbm.at[pl.ds(row0, CH)], bufs[0], sems[0])
    for k in range(NCH):
        cur = k % 2
        nxt = (k + 1) % 2
        handles[cur].wait()
        if k + 1 < NCH:
            handles[nxt] = pltpu.async_copy(
                x_hbm.at[pl.ds(row0 + (k + 1) * CH, CH)], bufs[nxt], sems[nxt]
            )
        xv = bufs[cur]

        def row_body(r, acc, xv=xv):
            sums, mns, mxs = acc
            new_s, new_n, new_x = [], [], []
            for g in range(G):
                v = xv[r, pl.ds(g * LANES, LANES)]
                new_s.append(sums[g] + v)
                new_n.append(jnp.minimum(mns[g], v))
                new_x.append(jnp.maximum(mxs[g], v))
            return (tuple(new_s), tuple(new_n), tuple(new_x))

        carry = lax.fori_loop(0, CH, row_body, carry)

    sums, mns, mxs = carry
    for g in range(G):
        pv[pl.ds(g * LANES, LANES)] = sums[g]
        pv[pl.ds(D + g * LANES, LANES)] = mns[g]
        pv[pl.ds(2 * D + g * LANES, LANES)] = mxs[g]
    pltpu.sync_copy(pv, part_hbm.at[w])


def _asm_kernel(inv_ref, part_ref, x_ref, out_ref):
    i = pl.program_id(0)
    p = part_ref[0]
    inv = inv_ref[i]
    mean = (p[0:1, 0:D] + p[1:2, 0:D]) * inv
    mn = jnp.minimum(p[0:1, D:2 * D], p[1:2, D:2 * D])
    mx = jnp.maximum(p[0:1, 2 * D:3 * D], p[1:2, 2 * D:3 * D])
    out_ref[:, 0:D] = jnp.broadcast_to(mean, (SEG, D))
    out_ref[:, D:2 * D] = jnp.broadcast_to(mn, (SEG, D))
    out_ref[:, 2 * D:3 * D] = jnp.broadcast_to(mx, (SEG, D))
    out_ref[:, 3 * D:4 * D] = x_ref[...]


def kernel(x_data, row_splits):
    counts = (row_splits[1:] - row_splits[:-1]).astype(jnp.float32)
    inv_counts = 1.0 / counts
    part = _sc_stats(x_data)
    part3 = part.reshape(B, 2, 3 * D)
    return pl.pallas_call(
        _asm_kernel,
        grid_spec=pltpu.PrefetchScalarGridSpec(
            num_scalar_prefetch=1,
            grid=(B,),
            in_specs=[
                pl.BlockSpec((1, 2, 3 * D), lambda i, *_: (i, 0, 0)),
                pl.BlockSpec((SEG, D), lambda i, *_: (i, 0)),
            ],
            out_specs=pl.BlockSpec((SEG, 4 * D), lambda i, *_: (i, 0)),
        ),
        out_shape=jax.ShapeDtypeStruct((TOTAL, 4 * D), jnp.float32),
    )(inv_counts, part3, x_data)


# speedup vs baseline: 1.2120x; 1.0169x over previous
import functools

import jax
import jax.numpy as jnp
from jax import lax
from jax.experimental import pallas as pl
from jax.experimental.pallas import tpu as pltpu
from jax.experimental.pallas import tpu_sc as plsc

B = 16
TOTAL = 16384
D = 256
SEG = TOTAL // B
HALF_B = B // 2
NC = 2
NS = 16
NW = NC * NS
WPS = NW // HALF_B
ROWS_W = SEG // WPS
CH = 128
NCH = ROWS_W // CH
LANES = 16
G = D // LANES

_sc_mesh = plsc.VectorSubcoreMesh(core_axis_name="c", subcore_axis_name="s")


def _make_sc_stats(seg_off):
    row_off = seg_off * SEG

    @functools.partial(
        pl.kernel,
        out_type=jax.ShapeDtypeStruct((HALF_B, WPS, 3 * D), jnp.float32),
        mesh=_sc_mesh,
        scratch_types=[
            pltpu.VMEM((CH, D), jnp.float32),
            pltpu.VMEM((CH, D), jnp.float32),
            pltpu.VMEM((3 * D,), jnp.float32),
            pltpu.SemaphoreType.DMA,
            pltpu.SemaphoreType.DMA,
        ],
    )
    def _sc_stats(x_hbm, part_hbm, xv0, xv1, pv, sem0, sem1):
        c = lax.axis_index("c")
        sub = lax.axis_index("s")
        w = c * NS + sub
        row0 = row_off + w * ROWS_W

        bufs = (xv0, xv1)
        sems = (sem0, sem1)

        zero = jnp.zeros((LANES,), jnp.float32)
        pinf = jnp.full((LANES,), jnp.inf, jnp.float32)
        ninf = jnp.full((LANES,), -jnp.inf, jnp.float32)
        carry = (
            tuple(zero for _ in range(G)),
            tuple(pinf for _ in range(G)),
            tuple(ninf for _ in range(G)),
        )

        handles = [None, None]
        handles[0] = pltpu.async_copy(x_hbm.at[pl.ds(row0, CH)], bufs[0], sems[0])
        for k in range(NCH):
            cur = k % 2
            nxt = (k + 1) % 2
            handles[cur].wait()
            if k + 1 < NCH:
                handles[nxt] = pltpu.async_copy(
                    x_hbm.at[pl.ds(row0 + (k + 1) * CH, CH)], bufs[nxt], sems[nxt]
                )
            xv = bufs[cur]

            def row_body(r, acc, xv=xv):
                sums, mns, mxs = acc
                new_s, new_n, new_x = [], [], []
                for g in range(G):
                    v = xv[r, pl.ds(g * LANES, LANES)]
                    new_s.append(sums[g] + v)
                    new_n.append(jnp.minimum(mns[g], v))
                    new_x.append(jnp.maximum(mxs[g], v))
                return (tuple(new_s), tuple(new_n), tuple(new_x))

            carry = lax.fori_loop(0, CH, row_body, carry)

        sums, mns, mxs = carry
        for g in range(G):
            pv[pl.ds(g * LANES, LANES)] = sums[g]
            pv[pl.ds(D + g * LANES, LANES)] = mns[g]
            pv[pl.ds(2 * D + g * LANES, LANES)] = mxs[g]
        pltpu.sync_copy(pv, part_hbm.at[w // WPS, w % WPS])

    return _sc_stats


_sc_stats_a = _make_sc_stats(0)
_sc_stats_b = _make_sc_stats(HALF_B)


def _asm_kernel(inv_ref, part_ref, x_ref, out_ref):
    i = pl.program_id(0)
    p = part_ref[0]
    inv = inv_ref[i]
    mean = jnp.sum(p[:, 0:D], axis=0, keepdims=True) * inv
    mn = jnp.min(p[:, D:2 * D], axis=0, keepdims=True)
    mx = jnp.max(p[:, 2 * D:3 * D], axis=0, keepdims=True)
    out_ref[:, 0:D] = jnp.broadcast_to(mean, (SEG, D))
    out_ref[:, D:2 * D] = jnp.broadcast_to(mn, (SEG, D))
    out_ref[:, 2 * D:3 * D] = jnp.broadcast_to(mx, (SEG, D))
    out_ref[:, 3 * D:4 * D] = x_ref[...]


def _asm_call(seg_off, inv_half, part, x_data, buf=None):
    in_specs = [
        pl.BlockSpec((1, WPS, 3 * D), lambda i, *_: (i, 0, 0)),
        pl.BlockSpec((SEG, D), lambda i, *_: (i + seg_off, 0)),
    ]
    operands = [inv_half, part, x_data]
    aliases = {}
    body = _asm_kernel
    if buf is not None:
        in_specs.append(pl.BlockSpec(memory_space=pltpu.MemorySpace.HBM))
        operands.append(buf)
        aliases = {3: 0}

        def body(inv_ref, part_ref, x_ref, buf_ref, out_ref):
            del buf_ref
            _asm_kernel(inv_ref, part_ref, x_ref, out_ref)

    return pl.pallas_call(
        body,
        grid_spec=pltpu.PrefetchScalarGridSpec(
            num_scalar_prefetch=1,
            grid=(HALF_B,),
            in_specs=in_specs,
            out_specs=pl.BlockSpec((SEG, 4 * D), lambda i, *_: (i + seg_off, 0)),
        ),
        out_shape=jax.ShapeDtypeStruct((TOTAL, 4 * D), jnp.float32),
        input_output_aliases=aliases,
    )(*operands)


def kernel(x_data, row_splits):
    counts = (row_splits[1:] - row_splits[:-1]).astype(jnp.float32)
    inv_counts = 1.0 / counts
    part_a = _sc_stats_a(x_data)
    part_b = _sc_stats_b(x_data)
    buf = _asm_call(0, inv_counts[0:HALF_B], part_a, x_data)
    return _asm_call(HALF_B, inv_counts[HALF_B:B], part_b, x_data, buf=buf)
